# speedup vs baseline: 1.8555x; 1.8555x over previous
import jax
import jax.numpy as jnp
from jax.experimental import pallas as pl
from jax.experimental.pallas import tpu as pltpu

N_TOK_BLK = 1024


def _router_body(x_ref, w_ref, g_ref, i_ref):
    xb = x_ref[...]
    w = w_ref[...]
    logits = jax.lax.dot_general(
        xb, w, (((1,), (1,)), ((), ())), preferred_element_type=jnp.float32
    )
    e = logits.shape[-1]
    iota = jax.lax.broadcasted_iota(jnp.int32, logits.shape, 1)
    m1 = jnp.max(logits, axis=-1, keepdims=True)
    i1 = jnp.min(jnp.where(logits == m1, iota, e), axis=-1, keepdims=True)
    masked = jnp.where(iota == i1, -jnp.inf, logits)
    m2 = jnp.max(masked, axis=-1, keepdims=True)
    i2 = jnp.min(jnp.where(masked == m2, iota, e), axis=-1, keepdims=True)
    t = jnp.exp(m2 - m1)
    g1 = 1.0 / (1.0 + t)
    g2 = t * g1
    g_ref[...] = jnp.concatenate([g1, g2], axis=-1)
    i_ref[...] = jnp.concatenate([i1, i2], axis=-1)


@jax.jit
def _router(x, W):
    n, d = x.shape
    num_e = W.shape[0]
    grid = (n // N_TOK_BLK,)
    gates, idx = pl.pallas_call(
        _router_body,
        grid=grid,
        in_specs=[
            pl.BlockSpec((N_TOK_BLK, d), lambda t: (t, 0)),
            pl.BlockSpec((num_e, d), lambda t: (0, 0)),
        ],
        out_specs=[
            pl.BlockSpec((N_TOK_BLK, 2), lambda t: (t, 0)),
            pl.BlockSpec((N_TOK_BLK, 2), lambda t: (t, 0)),
        ],
        out_shape=[
            jax.ShapeDtypeStruct((n, 2), jnp.float32),
            jax.ShapeDtypeStruct((n, 2), jnp.int32),
        ],
        compiler_params=pltpu.CompilerParams(
            dimension_semantics=("arbitrary",),
        ),
    )(x, W)
    return gates, idx


def kernel(x, W):
    gates, idx = _router(x, W)
    return gates, idx, jnp.zeros((), dtype=jnp.float32)


# speedup vs baseline: 1.8575x; 1.0011x over previous
import jax
import jax.numpy as jnp
from jax.experimental import pallas as pl
from jax.experimental.pallas import tpu as pltpu

N_TOK_BLK = 1024


def _router_body(x_ref, w_ref, g_ref, i_ref):
    xb = x_ref[...]
    w = w_ref[...]
    logits = jax.lax.dot_general(
        xb, w, (((1,), (1,)), ((), ())), preferred_element_type=jnp.float32
    )
    e = logits.shape[-1]
    iota = jax.lax.broadcasted_iota(jnp.int32, logits.shape, 1)
    m1 = jnp.max(logits, axis=-1, keepdims=True)
    i1 = jnp.min(jnp.where(logits == m1, iota, e), axis=-1, keepdims=True)
    masked = jnp.where(iota == i1, -jnp.inf, logits)
    m2 = jnp.max(masked, axis=-1, keepdims=True)
    i2 = jnp.min(jnp.where(masked == m2, iota, e), axis=-1, keepdims=True)
    t = jnp.exp(m2 - m1)
    g1 = 1.0 / (1.0 + t)
    g2 = t * g1
    g_ref[...] = jnp.concatenate([g1, g2], axis=-1)
    i_ref[...] = jnp.concatenate([i1, i2], axis=-1)


@jax.jit
def _router(x, W):
    n, d = x.shape
    num_e = W.shape[0]
    grid = (n // N_TOK_BLK,)
    gates, idx = pl.pallas_call(
        _router_body,
        grid=grid,
        in_specs=[
            pl.BlockSpec((N_TOK_BLK, d), lambda t: (t, 0)),
            pl.BlockSpec((num_e, d), lambda t: (0, 0)),
        ],
        out_specs=[
            pl.BlockSpec((N_TOK_BLK, 2), lambda t: (t, 0)),
            pl.BlockSpec((N_TOK_BLK, 2), lambda t: (t, 0)),
        ],
        out_shape=[
            jax.ShapeDtypeStruct((n, 2), jnp.float32),
            jax.ShapeDtypeStruct((n, 2), jnp.int32),
        ],
        compiler_params=pltpu.CompilerParams(
            dimension_semantics=("parallel",),
        ),
    )(x, W)
    return gates, idx


def kernel(x, W):
    gates, idx = _router(x, W)
    return gates, idx, jnp.zeros((), dtype=jnp.float32)
